# baseline (speedup 1.0000x reference)
import functools
import math

import jax
import jax.numpy as jnp
from jax import lax
from jax.experimental import pallas as pl
from jax.experimental.pallas import tpu as pltpu
from jax.experimental.pallas import tpu_sc as plsc

N = 10000
E = 320000
D = 128
NC = 2
NS = 16
NW = NC * NS
EPW = E // NW
K = 128
CH = 80
EPAD = CH * K
NP = 10240
SH = NP // NS

_mesh = plsc.VectorSubcoreMesh(core_axis_name="c", subcore_axis_name="s")


@functools.partial(
    pl.kernel,
    mesh=_mesh,
    out_type=jax.ShapeDtypeStruct((NC * NP,), jnp.float32),
    scratch_types=[
        pltpu.VMEM((CH, K), jnp.int32),
        pltpu.VMEM((K,), jnp.float32),
        pltpu.VMEM((SH,), jnp.float32),
        pltpu.VMEM_SHARED((NP,), jnp.float32),
    ],
)
def _deg_kernel(dst3, ones_hbm, zeros_hbm, out, dst_v, ones_v, bounce_v,
                deg_sh):
    c = lax.axis_index("c")
    s = lax.axis_index("s")
    w = c * NS + s
    pltpu.sync_copy(dst3.at[w], dst_v)
    pltpu.sync_copy(ones_hbm, ones_v)
    pltpu.sync_copy(zeros_hbm.at[pl.ds(s * SH, SH)], bounce_v)
    pltpu.sync_copy(bounce_v, deg_sh.at[pl.ds(s * SH, SH)])
    plsc.subcore_barrier()

    def body(j, carry):
        pltpu.sync_copy(ones_v, deg_sh.at[dst_v.at[j]], add=True)
        return carry

    lax.fori_loop(0, CH, body, 0)
    plsc.subcore_barrier()
    pltpu.sync_copy(deg_sh.at[pl.ds(s * SH, SH)], bounce_v)
    pltpu.sync_copy(bounce_v, out.at[pl.ds(c * NP + s * SH, SH)])


GRP = 8
DH = D // NC
CH2 = 160
NGRP2 = CH2 // GRP
EPT = CH2 * K
NG = N + 8
SH2 = 624
G_TAIL = NG - NS * SH2
O_TAIL = N - NS * SH2


@functools.partial(
    pl.kernel,
    mesh=_mesh,
    out_type=jax.ShapeDtypeStruct((NC, N, DH), jnp.float32),
    scratch_types=[
        pltpu.VMEM((GRP, K), jnp.int32),
        pltpu.VMEM((GRP, K), jnp.int32),
        pltpu.VMEM((K,), jnp.int32),
        pltpu.VMEM((K, DH), jnp.float32),
        pltpu.VMEM((K, DH), jnp.float32),
        pltpu.VMEM((K, DH), jnp.float32),
        pltpu.VMEM_SHARED((NG, DH), jnp.float32),
        pltpu.VMEM_SHARED((N, DH), jnp.float32),
        pltpu.SemaphoreType.DMA,
        pltpu.SemaphoreType.DMA,
        pltpu.SemaphoreType.DMA,
    ],
    compiler_params=pltpu.CompilerParams(use_tc_tiling_on_sc=False),
)
def _edge_kernel(src3, dst3, srch4, gflat, zeros_hbm, out,
                 src_gv, dst_gv, idxh_v, rows0, rows1, rows_h,
                 g_sh, out_sh, sem0, sem1, sem_h):
    c = lax.axis_index("c")
    s = lax.axis_index("s")
    pltpu.sync_copy(gflat.at[pl.ds(c * NG + s * SH2, SH2)],
                    g_sh.at[pl.ds(s * SH2, SH2)])
    pltpu.sync_copy(zeros_hbm.at[pl.ds(s * SH2, SH2)],
                    out_sh.at[pl.ds(s * SH2, SH2)])

    @pl.when(s == 0)
    def _tails():
        pltpu.sync_copy(gflat.at[pl.ds(c * NG + NS * SH2, G_TAIL)],
                        g_sh.at[pl.ds(NS * SH2, G_TAIL)])
        pltpu.sync_copy(zeros_hbm.at[pl.ds(NS * SH2, O_TAIL)],
                        out_sh.at[pl.ds(NS * SH2, O_TAIL)])

    plsc.subcore_barrier()

    rows = (rows0, rows1)
    sems = (sem0, sem1)

    def group(g, carry):
        pltpu.sync_copy(src3.at[s, pl.ds(g * GRP, GRP)], src_gv)
        pltpu.sync_copy(dst3.at[s, pl.ds(g * GRP, GRP)], dst_gv)
        pltpu.sync_copy(srch4.at[c, s, g], idxh_v)
        hh = pltpu.async_copy(gflat.at[idxh_v], rows_h, sem_h)
        handles = [
            pltpu.async_copy(g_sh.at[src_gv.at[b]], rows[b], sems[b])
            for b in range(2)
        ]
        for b in range(GRP - 1):
            p = b % 2
            handles[p].wait()
            pltpu.sync_copy(rows[p], out_sh.at[dst_gv.at[b]], add=True)
            if b + 2 < GRP - 1:
                handles[p] = pltpu.async_copy(
                    g_sh.at[src_gv.at[b + 2]], rows[p], sems[p])
        hh.wait()
        pltpu.sync_copy(rows_h, out_sh.at[dst_gv.at[GRP - 1]], add=True)
        return carry

    lax.fori_loop(0, NGRP2, group, 0)
    plsc.subcore_barrier()
    pltpu.sync_copy(out_sh.at[pl.ds(s * SH2, SH2)],
                    out.at[c, pl.ds(s * SH2, SH2)])

    @pl.when(s == 0)
    def _wb_tail():
        pltpu.sync_copy(out_sh.at[pl.ds(NS * SH2, O_TAIL)],
                        out.at[c, pl.ds(NS * SH2, O_TAIL)])


def _mm_body(x_ref, w_ref, h_ref):
    h_ref[...] = jnp.dot(x_ref[...], w_ref[...],
                         preferred_element_type=jnp.float32)


_mm = pl.pallas_call(
    _mm_body,
    out_shape=jax.ShapeDtypeStruct((N, D), jnp.float32),
)


def _scale_body(h_ref, deg2_ref, gs_ref, dinv_ref):
    deg = deg2_ref[0, :N] + deg2_ref[1, :N] + 1.0
    dinv = lax.rsqrt(deg)
    hs = h_ref[...] * dinv
    gs_ref[0, :N] = hs[:, :DH]
    gs_ref[0, N:] = jnp.zeros((NG - N, DH), jnp.float32)
    gs_ref[1, :N] = hs[:, DH:]
    gs_ref[1, N:] = jnp.zeros((NG - N, DH), jnp.float32)
    dinv_ref[...] = dinv


_scale = pl.pallas_call(
    _scale_body,
    out_shape=(jax.ShapeDtypeStruct((NC, NG, DH), jnp.float32),
               jax.ShapeDtypeStruct((N, 1), jnp.float32)),
)


def _post_body(raw_ref, gs_ref, dinv_ref, x_ref, b_ref, gam_ref, bet_ref,
               o_ref):
    t = jnp.concatenate([raw_ref[0] + gs_ref[0, :N],
                         raw_ref[1] + gs_ref[1, :N]], axis=1)
    o = t * dinv_ref[...] + b_ref[...]
    mean = jnp.mean(o, axis=0, keepdims=True)
    cen = o - mean
    var = jnp.mean(cen * cen, axis=0, keepdims=True)
    o = cen * lax.rsqrt(var + 1e-5) * gam_ref[...] + bet_ref[...]
    o = 0.5 * o * (1.0 + lax.erf(o * (1.0 / math.sqrt(2.0))))
    o_ref[...] = o + x_ref[...]


_post = pl.pallas_call(
    _post_body,
    out_shape=jax.ShapeDtypeStruct((N, D), jnp.float32),
)


def kernel(x, edge_index, W, b, gamma, beta):
    dst2 = edge_index[1].reshape(NW, EPW)
    dst3 = jnp.pad(dst2, ((0, 0), (0, EPAD - EPW)),
                   constant_values=N).reshape(NW, CH, K)
    ones1 = jnp.ones((K,), jnp.float32)
    zeros1 = jnp.zeros((NP,), jnp.float32)
    deg_raw = _deg_kernel(dst3, ones1, zeros1)
    h = _mm(x, W)
    gsplit, dinv = _scale(h, deg_raw.reshape(NC, NP, 1))
    src2e = edge_index[0].reshape(NS, E // NS)
    dst2e = edge_index[1].reshape(NS, E // NS)
    src3e = jnp.pad(src2e, ((0, 0), (0, EPT - E // NS)),
                    constant_values=N).reshape(NS, CH2, K)
    dst3e = jnp.pad(dst2e, ((0, 0), (0, EPT - E // NS))).reshape(NS, CH2, K)
    srch4 = (src3e[None, :, GRP - 1::GRP, :]
             + (jnp.arange(NC, dtype=jnp.int32) * NG)[:, None, None, None])
    zerosH = jnp.zeros((N, DH), jnp.float32)
    raw = _edge_kernel(src3e, dst3e, srch4, gsplit.reshape(NC * NG, DH),
                       zerosH)
    return _post(raw, gsplit, dinv, x, b.reshape(1, D), gamma.reshape(1, D),
                 beta.reshape(1, D))


# speedup vs baseline: 1.1993x; 1.1993x over previous
import functools
import math

import jax
import jax.numpy as jnp
from jax import lax
from jax.experimental import pallas as pl
from jax.experimental.pallas import tpu as pltpu
from jax.experimental.pallas import tpu_sc as plsc

N = 10000
E = 320000
D = 128
NC = 2
NS = 16
NW = NC * NS
EPW = E // NW
K = 128
CH = 80
EPAD = CH * K
NP = 10240
SH = NP // NS

_mesh = plsc.VectorSubcoreMesh(core_axis_name="c", subcore_axis_name="s")


@functools.partial(
    pl.kernel,
    mesh=_mesh,
    out_type=jax.ShapeDtypeStruct((NC * NP,), jnp.float32),
    scratch_types=[
        pltpu.VMEM((CH, K), jnp.int32),
        pltpu.VMEM((K,), jnp.float32),
        pltpu.VMEM_SHARED((NP,), jnp.float32),
    ],
    compiler_params=pltpu.CompilerParams(use_tc_tiling_on_sc=False),
)
def _deg_kernel(dst3, ones_hbm, zeros_hbm, out, dst_v, ones_v, deg_sh):
    c = lax.axis_index("c")
    s = lax.axis_index("s")
    w = c * NS + s
    pltpu.sync_copy(dst3.at[w], dst_v)
    pltpu.sync_copy(ones_hbm, ones_v)
    pltpu.sync_copy(zeros_hbm.at[pl.ds(s * SH, SH)],
                    deg_sh.at[pl.ds(s * SH, SH)])
    plsc.subcore_barrier()

    def body(j, carry):
        pltpu.sync_copy(ones_v, deg_sh.at[dst_v.at[j]], add=True)
        return carry

    lax.fori_loop(0, CH, body, 0)
    plsc.subcore_barrier()
    pltpu.sync_copy(deg_sh.at[pl.ds(s * SH, SH)],
                    out.at[pl.ds(c * NP + s * SH, SH)])


GRP = 32
DH = D // NC
CH2 = 160
NGRP2 = CH2 // GRP
EPT = CH2 * K
NG = N + 8
SH2 = 624
G_TAIL = NG - NS * SH2
O_TAIL = N - NS * SH2


@functools.partial(
    pl.kernel,
    mesh=_mesh,
    out_type=jax.ShapeDtypeStruct((NC, N, DH), jnp.float32),
    scratch_types=[
        pltpu.VMEM((GRP, K), jnp.int32),
        pltpu.VMEM((GRP, K), jnp.int32),
        pltpu.VMEM((K, DH), jnp.float32),
        pltpu.VMEM((K, DH), jnp.float32),
        pltpu.VMEM((K, DH), jnp.float32),
        pltpu.VMEM((K, DH), jnp.float32),
        pltpu.VMEM_SHARED((NG, DH), jnp.float32),
        pltpu.VMEM_SHARED((N, DH), jnp.float32),
        pltpu.SemaphoreType.DMA,
        pltpu.SemaphoreType.DMA,
        pltpu.SemaphoreType.DMA,
        pltpu.SemaphoreType.DMA,
    ],
    compiler_params=pltpu.CompilerParams(use_tc_tiling_on_sc=False),
)
def _edge_kernel(src3, dst3, gflat, zeros_hbm, out,
                 src_gv, dst_gv, rows0, rows1, rows2, rows3, g_sh, out_sh,
                 sem0, sem1, sem2, sem3):
    c = lax.axis_index("c")
    s = lax.axis_index("s")
    pltpu.sync_copy(gflat.at[pl.ds(c * NG + s * SH2, SH2)],
                    g_sh.at[pl.ds(s * SH2, SH2)])
    pltpu.sync_copy(zeros_hbm.at[pl.ds(s * SH2, SH2)],
                    out_sh.at[pl.ds(s * SH2, SH2)])

    @pl.when(s == 0)
    def _tails():
        pltpu.sync_copy(gflat.at[pl.ds(c * NG + NS * SH2, G_TAIL)],
                        g_sh.at[pl.ds(NS * SH2, G_TAIL)])
        pltpu.sync_copy(zeros_hbm.at[pl.ds(NS * SH2, O_TAIL)],
                        out_sh.at[pl.ds(NS * SH2, O_TAIL)])

    plsc.subcore_barrier()

    rows = (rows0, rows1, rows2, rows3)
    sems = (sem0, sem1, sem2, sem3)
    NB = 4

    def group(g, carry):
        pltpu.sync_copy(src3.at[s, pl.ds(g * GRP, GRP)], src_gv)
        pltpu.sync_copy(dst3.at[s, pl.ds(g * GRP, GRP)], dst_gv)
        handles = [
            pltpu.async_copy(g_sh.at[src_gv.at[b]], rows[b], sems[b])
            for b in range(NB)
        ]
        for b in range(GRP):
            p = b % NB
            handles[p].wait()
            pltpu.sync_copy(rows[p], out_sh.at[dst_gv.at[b]], add=True)
            if b + NB < GRP:
                handles[p] = pltpu.async_copy(
                    g_sh.at[src_gv.at[b + NB]], rows[p], sems[p])
        return carry

    lax.fori_loop(0, NGRP2, group, 0)
    plsc.subcore_barrier()
    pltpu.sync_copy(out_sh.at[pl.ds(s * SH2, SH2)],
                    out.at[c, pl.ds(s * SH2, SH2)])

    @pl.when(s == 0)
    def _wb_tail():
        pltpu.sync_copy(out_sh.at[pl.ds(NS * SH2, O_TAIL)],
                        out.at[c, pl.ds(NS * SH2, O_TAIL)])


def _scale_body(x_ref, w_ref, deg2_ref, gs_ref, dinv_ref):
    deg = deg2_ref[0, :N] + deg2_ref[1, :N] + 1.0
    dinv = lax.rsqrt(deg)
    h = jnp.dot(x_ref[...], w_ref[...], preferred_element_type=jnp.float32)
    hs = h * dinv
    gs_ref[0, :N] = hs[:, :DH]
    gs_ref[0, N:] = jnp.zeros((NG - N, DH), jnp.float32)
    gs_ref[1, :N] = hs[:, DH:]
    gs_ref[1, N:] = jnp.zeros((NG - N, DH), jnp.float32)
    dinv_ref[...] = dinv


_scale = pl.pallas_call(
    _scale_body,
    out_shape=(jax.ShapeDtypeStruct((NC, NG, DH), jnp.float32),
               jax.ShapeDtypeStruct((N, 1), jnp.float32)),
)


def _post_body(raw_ref, gs_ref, dinv_ref, x_ref, b_ref, gam_ref, bet_ref,
               o_ref):
    t = jnp.concatenate([raw_ref[0] + gs_ref[0, :N],
                         raw_ref[1] + gs_ref[1, :N]], axis=1)
    o = t * dinv_ref[...] + b_ref[...]
    mean = jnp.mean(o, axis=0, keepdims=True)
    cen = o - mean
    var = jnp.mean(cen * cen, axis=0, keepdims=True)
    o = cen * lax.rsqrt(var + 1e-5) * gam_ref[...] + bet_ref[...]
    o = 0.5 * o * (1.0 + lax.erf(o * (1.0 / math.sqrt(2.0))))
    o_ref[...] = o + x_ref[...]


_post = pl.pallas_call(
    _post_body,
    out_shape=jax.ShapeDtypeStruct((N, D), jnp.float32),
)


def kernel(x, edge_index, W, b, gamma, beta):
    dst2 = edge_index[1].reshape(NW, EPW)
    dst3 = jnp.pad(dst2, ((0, 0), (0, EPAD - EPW)),
                   constant_values=N).reshape(NW, CH, K)
    ones1 = jnp.ones((K,), jnp.float32)
    zeros1 = jnp.zeros((NP,), jnp.float32)
    deg_raw = _deg_kernel(dst3, ones1, zeros1)
    gsplit, dinv = _scale(x, W, deg_raw.reshape(NC, NP, 1))
    src2e = edge_index[0].reshape(NS, E // NS)
    dst2e = edge_index[1].reshape(NS, E // NS)
    src3e = jnp.pad(src2e, ((0, 0), (0, EPT - E // NS)),
                    constant_values=N).reshape(NS, CH2, K)
    dst3e = jnp.pad(dst2e, ((0, 0), (0, EPT - E // NS))).reshape(NS, CH2, K)
    zerosH = jnp.zeros((N, DH), jnp.float32)
    raw = _edge_kernel(src3e, dst3e, gsplit.reshape(NC * NG, DH), zerosH)
    return _post(raw, gsplit, dinv, x, b.reshape(1, D), gamma.reshape(1, D),
                 beta.reshape(1, D))


# speedup vs baseline: 1.2220x; 1.0189x over previous
import functools
import math

import jax
import jax.numpy as jnp
from jax import lax
from jax.experimental import pallas as pl
from jax.experimental.pallas import tpu as pltpu
from jax.experimental.pallas import tpu_sc as plsc

N = 10000
E = 320000
D = 128
NC = 2
NS = 16
NW = NC * NS
EPW = E // NW
K = 128
CH = 80
EPAD = CH * K
NP = 10240
SH = NP // NS

_mesh = plsc.VectorSubcoreMesh(core_axis_name="c", subcore_axis_name="s")


@functools.partial(
    pl.kernel,
    mesh=_mesh,
    out_type=jax.ShapeDtypeStruct((NC * NP,), jnp.float32),
    scratch_types=[
        pltpu.VMEM((CH, K), jnp.int32),
        pltpu.VMEM((K,), jnp.float32),
        pltpu.VMEM_SHARED((NP,), jnp.float32),
    ],
    compiler_params=pltpu.CompilerParams(use_tc_tiling_on_sc=False),
)
def _deg_kernel(dst3, ones_hbm, zeros_hbm, out, dst_v, ones_v, deg_sh):
    c = lax.axis_index("c")
    s = lax.axis_index("s")
    w = c * NS + s
    pltpu.sync_copy(dst3.at[w], dst_v)
    pltpu.sync_copy(ones_hbm, ones_v)
    pltpu.sync_copy(zeros_hbm.at[pl.ds(s * SH, SH)],
                    deg_sh.at[pl.ds(s * SH, SH)])
    plsc.subcore_barrier()

    def body(j, carry):
        pltpu.sync_copy(ones_v, deg_sh.at[dst_v.at[j]], add=True)
        return carry

    lax.fori_loop(0, CH, body, 0)
    plsc.subcore_barrier()
    pltpu.sync_copy(deg_sh.at[pl.ds(s * SH, SH)],
                    out.at[pl.ds(c * NP + s * SH, SH)])


GRP = 32
DH = D // NC
CH2 = 160
NGRP2 = CH2 // GRP
EPT = CH2 * K
NG = N + 8
SH2 = 624
G_TAIL = NG - NS * SH2
O_TAIL = N - NS * SH2


@functools.partial(
    pl.kernel,
    mesh=_mesh,
    out_type=jax.ShapeDtypeStruct((NC, N, DH), jnp.float32),
    scratch_types=[
        pltpu.VMEM((GRP, K), jnp.int32),
        pltpu.VMEM((GRP, K), jnp.int32),
        pltpu.VMEM((K, DH), jnp.float32),
        pltpu.VMEM((K, DH), jnp.float32),
        pltpu.VMEM((K, DH), jnp.float32),
        pltpu.VMEM((K, DH), jnp.float32),
        pltpu.VMEM_SHARED((NG, DH), jnp.float32),
        pltpu.VMEM_SHARED((N, DH), jnp.float32),
        pltpu.SemaphoreType.DMA,
        pltpu.SemaphoreType.DMA,
        pltpu.SemaphoreType.DMA,
        pltpu.SemaphoreType.DMA,
    ],
    compiler_params=pltpu.CompilerParams(use_tc_tiling_on_sc=False),
)
def _edge_kernel(src3, dst3, gflat, zeros_hbm, out,
                 src_gv, dst_gv, rows0, rows1, rows2, rows3, g_sh, out_sh,
                 sem0, sem1, sem2, sem3):
    c = lax.axis_index("c")
    s = lax.axis_index("s")
    a_g = pltpu.async_copy(gflat.at[pl.ds(c * NG + s * SH2, SH2)],
                           g_sh.at[pl.ds(s * SH2, SH2)], sem0)
    a_z = pltpu.async_copy(zeros_hbm.at[pl.ds(s * SH2, SH2)],
                           out_sh.at[pl.ds(s * SH2, SH2)], sem1)

    @pl.when(s == 0)
    def _tails():
        t_g = pltpu.async_copy(gflat.at[pl.ds(c * NG + NS * SH2, G_TAIL)],
                               g_sh.at[pl.ds(NS * SH2, G_TAIL)], sem2)
        t_z = pltpu.async_copy(zeros_hbm.at[pl.ds(NS * SH2, O_TAIL)],
                               out_sh.at[pl.ds(NS * SH2, O_TAIL)], sem3)
        t_g.wait()
        t_z.wait()

    a_g.wait()
    a_z.wait()
    plsc.subcore_barrier()

    rows = (rows0, rows1, rows2, rows3)
    sems = (sem0, sem1, sem2, sem3)
    NB = 4

    def group(g, carry):
        i_s = pltpu.async_copy(src3.at[s, pl.ds(g * GRP, GRP)], src_gv, sem0)
        i_d = pltpu.async_copy(dst3.at[s, pl.ds(g * GRP, GRP)], dst_gv, sem1)
        i_s.wait()
        i_d.wait()
        handles = [
            pltpu.async_copy(g_sh.at[src_gv.at[b]], rows[b], sems[b])
            for b in range(NB)
        ]
        for b in range(GRP):
            p = b % NB
            handles[p].wait()
            pltpu.sync_copy(rows[p], out_sh.at[dst_gv.at[b]], add=True)
            if b + NB < GRP:
                handles[p] = pltpu.async_copy(
                    g_sh.at[src_gv.at[b + NB]], rows[p], sems[p])
        return carry

    lax.fori_loop(0, NGRP2, group, 0)
    plsc.subcore_barrier()
    pltpu.sync_copy(out_sh.at[pl.ds(s * SH2, SH2)],
                    out.at[c, pl.ds(s * SH2, SH2)])

    @pl.when(s == 0)
    def _wb_tail():
        pltpu.sync_copy(out_sh.at[pl.ds(NS * SH2, O_TAIL)],
                        out.at[c, pl.ds(NS * SH2, O_TAIL)])


def _scale_body(x_ref, w_ref, deg2_ref, gs_ref, dinv_ref):
    deg = deg2_ref[0, :N] + deg2_ref[1, :N] + 1.0
    dinv = lax.rsqrt(deg)
    h = jnp.dot(x_ref[...], w_ref[...], preferred_element_type=jnp.float32)
    hs = h * dinv
    gs_ref[0, :N] = hs[:, :DH]
    gs_ref[0, N:] = jnp.zeros((NG - N, DH), jnp.float32)
    gs_ref[1, :N] = hs[:, DH:]
    gs_ref[1, N:] = jnp.zeros((NG - N, DH), jnp.float32)
    dinv_ref[...] = dinv


_scale = pl.pallas_call(
    _scale_body,
    out_shape=(jax.ShapeDtypeStruct((NC, NG, DH), jnp.float32),
               jax.ShapeDtypeStruct((N, 1), jnp.float32)),
)


def _post_body(raw_ref, gs_ref, dinv_ref, x_ref, b_ref, gam_ref, bet_ref,
               o_ref):
    t = jnp.concatenate([raw_ref[0] + gs_ref[0, :N],
                         raw_ref[1] + gs_ref[1, :N]], axis=1)
    o = t * dinv_ref[...] + b_ref[...]
    mean = jnp.mean(o, axis=0, keepdims=True)
    cen = o - mean
    var = jnp.mean(cen * cen, axis=0, keepdims=True)
    o = cen * lax.rsqrt(var + 1e-5) * gam_ref[...] + bet_ref[...]
    o = 0.5 * o * (1.0 + lax.erf(o * (1.0 / math.sqrt(2.0))))
    o_ref[...] = o + x_ref[...]


_post = pl.pallas_call(
    _post_body,
    out_shape=jax.ShapeDtypeStruct((N, D), jnp.float32),
)


def kernel(x, edge_index, W, b, gamma, beta):
    dst2 = edge_index[1].reshape(NW, EPW)
    dst3 = jnp.pad(dst2, ((0, 0), (0, EPAD - EPW)),
                   constant_values=N).reshape(NW, CH, K)
    ones1 = jnp.ones((K,), jnp.float32)
    zeros1 = jnp.zeros((NP,), jnp.float32)
    deg_raw = _deg_kernel(dst3, ones1, zeros1)
    gsplit, dinv = _scale(x, W, deg_raw.reshape(NC, NP, 1))
    src2e = edge_index[0].reshape(NS, E // NS)
    dst2e = edge_index[1].reshape(NS, E // NS)
    src3e = jnp.pad(src2e, ((0, 0), (0, EPT - E // NS)),
                    constant_values=N).reshape(NS, CH2, K)
    dst3e = jnp.pad(dst2e, ((0, 0), (0, EPT - E // NS))).reshape(NS, CH2, K)
    zerosH = jnp.zeros((N, DH), jnp.float32)
    raw = _edge_kernel(src3e, dst3e, gsplit.reshape(NC * NG, DH), zerosH)
    return _post(raw, gsplit, dinv, x, b.reshape(1, D), gamma.reshape(1, D),
                 beta.reshape(1, D))
